# baseline (speedup 1.0000x reference)
import functools

import jax
import jax.numpy as jnp
from jax import lax
from jax.experimental import pallas as pl
from jax.experimental.pallas import tpu as pltpu
from jax.experimental.pallas import tpu_sc as plsc

B, P, D = 64, 8192, 128
K = 64
R, C = 64, 128
NEG_INF = float("-inf")
POS_INF = float("inf")


def _sim_topk_body(ln_ref, pn_ref, ti_ref, bi_ref):
    lb = ln_ref[0].astype(jnp.bfloat16)
    pb = pn_ref[0].astype(jnp.bfloat16)
    s = lax.dot_general(pb, lb, (((1,), (1,)), ((), ())),
                        preferred_element_type=jnp.float32)
    sm = s.reshape(R, C)

    lane = lax.broadcasted_iota(jnp.int32, (R, C), 1)
    row = lax.broadcasted_iota(jnp.int32, (R, C), 0)
    flat = row * C + lane
    out_iota = lax.broadcasted_iota(jnp.int32, (1, K), 1)

    def rnd(r, carry):
        wt, wb, ti, bi = carry
        mt = jnp.max(wt)
        post = jnp.min(jnp.where(wt == mt, flat, P))
        ti = jnp.where(out_iota == r, post, ti)
        wt = jnp.where(flat == post, NEG_INF, wt)
        mb = jnp.min(wb)
        posb = jnp.min(jnp.where(wb == mb, flat, P))
        bi = jnp.where(out_iota == r, posb, bi)
        wb = jnp.where(flat == posb, POS_INF, wb)
        return wt, wb, ti, bi

    init = (sm, sm, jnp.zeros((1, K), jnp.int32), jnp.zeros((1, K), jnp.int32))
    _, _, ti, bi = lax.fori_loop(0, K, rnd, init)
    ti_ref[...] = ti.reshape(1, 1, K)
    bi_ref[...] = bi.reshape(1, 1, K)


def _compute_topk_idx(ln, pn):
    return pl.pallas_call(
        _sim_topk_body,
        grid=(B,),
        in_specs=[
            pl.BlockSpec((1, P, D), lambda b: (b, 0, 0)),
            pl.BlockSpec((1, 1, D), lambda b: (b, 0, 0)),
        ],
        out_specs=[
            pl.BlockSpec((1, 1, K), lambda b: (b, 0, 0)),
            pl.BlockSpec((1, 1, K), lambda b: (b, 0, 0)),
        ],
        out_shape=[
            jax.ShapeDtypeStruct((B, 1, K), jnp.int32),
            jax.ShapeDtypeStruct((B, 1, K), jnp.int32),
        ],
    )(ln, pn)


def _make_gather_kernel():
    info = plsc.get_sparse_core_info()
    nc = info.num_cores
    mesh = plsc.VectorSubcoreMesh(core_axis_name="c", subcore_axis_name="s")

    @functools.partial(
        pl.kernel,
        out_type=(
            jax.ShapeDtypeStruct((B, K, D), jnp.float32),
            jax.ShapeDtypeStruct((B, K, D), jnp.float32),
        ),
        mesh=mesh,
        scratch_types=[
            pltpu.VMEM((K,), jnp.int32),
            pltpu.VMEM((K,), jnp.int32),
            pltpu.VMEM((16,), jnp.int32),
            pltpu.VMEM((K, D), jnp.float32),
            pltpu.SemaphoreType.DMA,
        ],
    )
    def gather_kernel(ti_hbm, bi_hbm, base_hbm, feat_hbm, tf_hbm, bf_hbm,
                      idxv, gidx, bv, rows, sem):
        w = lax.axis_index("s") * nc + lax.axis_index("c")
        for bi_ in range(B // 32):
            b = w * (B // 32) + bi_
            pltpu.sync_copy(base_hbm.at[b], bv)
            base16 = bv[...]
            for side in range(2):
                src = ti_hbm if side == 0 else bi_hbm
                dst = tf_hbm if side == 0 else bf_hbm
                pltpu.sync_copy(src.at[b], idxv)
                for j in range(K // 16):
                    gidx[pl.ds(j * 16, 16)] = idxv[pl.ds(j * 16, 16)] + base16
                pltpu.async_copy(feat_hbm.at[gidx], rows, sem).wait()
                pltpu.sync_copy(rows, dst.at[b])

    return gather_kernel


_gather_kernel = _make_gather_kernel()


def kernel(local_features, prototypes):
    ln = local_features / (jnp.linalg.norm(local_features, axis=-1, keepdims=True) + 1e-08)
    pn = prototypes / (jnp.linalg.norm(prototypes, axis=-1, keepdims=True) + 1e-08)
    ti3, bi3 = _compute_topk_idx(ln, pn)
    ti = ti3.reshape(B, K)
    bi = bi3.reshape(B, K)
    bases = jnp.broadcast_to((jnp.arange(B, dtype=jnp.int32) * P)[:, None], (B, 16))
    featrows = local_features.reshape(B * P, D)
    tf, bf = _gather_kernel(ti, bi, bases, featrows)
    return tf, bf, ti, bi
